# baseline (speedup 1.0000x reference)
import functools

import jax
import jax.numpy as jnp
from jax import lax
from jax.experimental import pallas as pl
from jax.experimental.pallas import tpu as pltpu, tpu_sc as plsc

_DIM = 240
_NSEG = 112
_N = 100000
_NW = 32
_C = 128
_NFULL = _N // _C
_TAIL = _N - _NFULL * _C
_TAIL_W = 31


def _sqrt16_batch(ss):
    ys = [plsc.bitcast(jnp.int32(0x5F3759DF)
                       - (plsc.bitcast(s, jnp.int32) >> 1), jnp.float32)
          for s in ss]
    hs = [0.5 * s for s in ss]
    for _ in range(2):
        ys = [y * (1.5 - h * y * y) for y, h in zip(ys, hs)]
    return [s * y for s, y in zip(ss, ys)]


def _colvec(xb, ob, off):
    for j0 in range(0, 64, 8):
        vs = [xb[j0 + j, pl.ds(off, 16)] for j in range(8)]
        for j in range(8):
            ob[j0 + j, pl.ds(off, 16)] = jnp.abs(vs[j])
    for k0 in range(0, 32, 8):
        gs = [[xb[64 + 3 * (k0 + k) + t, pl.ds(off, 16)] for t in range(3)]
              for k in range(8)]
        sq = [[g * g for g in row] for row in gs]
        ss = [r[0] + r[1] + r[2] for r in sq]
        rs = _sqrt16_batch(ss)
        for k in range(8):
            ob[64 + k0 + k, pl.ds(off, 16)] = rs[k]
    for k0 in range(0, 16, 8):
        gs = [[xb[160 + 5 * (k0 + k) + t, pl.ds(off, 16)] for t in range(5)]
              for k in range(8)]
        sq = [[g * g for g in row] for row in gs]
        ss = [((r[0] + r[1]) + (r[2] + r[3])) + r[4] for r in sq]
        rs = _sqrt16_batch(ss)
        for k in range(8):
            ob[96 + k0 + k, pl.ds(off, 16)] = rs[k]


def _body(x_hbm, xt_hbm, o_hbm, ot_hbm, xbuf, obuf, insem, outsem):
    wid = lax.axis_index("s") * 2 + lax.axis_index("c")
    nck = jnp.where(wid <= 12, 25, 24)

    def in_copy(k, slot):
        col0 = (wid + k * _NW) * _C
        return pltpu.make_async_copy(
            x_hbm.at[:, pl.ds(col0, _C)],
            xbuf.at[:, pl.ds(slot * _C, _C)],
            insem.at[slot])

    def out_copy(k, slot):
        col0 = (wid + k * _NW) * _C
        return pltpu.make_async_copy(
            obuf.at[:, pl.ds(slot * _C, _C)],
            o_hbm.at[:, pl.ds(col0, _C)],
            outsem.at[slot])

    in_copy(0, 0).start()

    @pl.loop(0, nck)
    def _chunk(k):
        slot = lax.rem(k, 2)

        @pl.when(k + 1 < nck)
        def _prefetch():
            in_copy(k + 1, 1 - slot).start()

        in_copy(k, slot).wait()

        @pl.when(k >= 2)
        def _drain():
            out_copy(k - 2, slot).wait()

        base = slot * _C

        @plsc.parallel_loop(0, _C // 16)
        def _cv(v):
            _colvec(xbuf, obuf, base + v * 16)

        out_copy(k, slot).start()

    out_copy(nck - 2, lax.rem(nck - 2, 2)).wait()
    out_copy(nck - 1, lax.rem(nck - 1, 2)).wait()

    @pl.when(wid == _TAIL_W)
    def _tail():
        pltpu.sync_copy(xt_hbm, xbuf.at[:, pl.ds(0, _C)])

        @plsc.parallel_loop(0, _TAIL // 16)
        def _cv(v):
            _colvec(xbuf, obuf, v * 16)

        pltpu.sync_copy(obuf.at[:, pl.ds(0, _C)], ot_hbm)


def kernel(features):
    size = features.shape[:-1]
    x = features.reshape(-1, _DIM)
    xt = x.T
    x_tail = jnp.pad(lax.slice(x, (_NFULL * _C, 0), (_N, _DIM)).T,
                     ((0, 0), (0, _C - _TAIL)))
    mesh = plsc.VectorSubcoreMesh(core_axis_name="c", subcore_axis_name="s",
                                  num_cores=2, num_subcores=16)
    out_t, out_tail = pl.kernel(
        _body,
        out_type=[
            jax.ShapeDtypeStruct((_NSEG, _N), jnp.float32),
            jax.ShapeDtypeStruct((_NSEG, _C), jnp.float32),
        ],
        mesh=mesh,
        scratch_types=[
            pltpu.VMEM((_DIM, 2 * _C), jnp.float32),
            pltpu.VMEM((_NSEG, 2 * _C), jnp.float32),
            pltpu.SemaphoreType.DMA((2,)),
            pltpu.SemaphoreType.DMA((2,)),
        ],
        compiler_params=pltpu.CompilerParams(needs_layout_passes=False),
    )(xt, x_tail)
    tail = lax.slice(out_tail, (0, 0), (_NSEG, _TAIL)).T
    out = lax.dynamic_update_slice(out_t.T, tail, (_NFULL * _C, 0))
    return out.reshape(size + (_NSEG,))


# speedup vs baseline: 1.0009x; 1.0009x over previous
import jax
import jax.numpy as jnp
from jax import lax
from jax.experimental import pallas as pl
from jax.experimental.pallas import tpu as pltpu, tpu_sc as plsc

_DIM = 240
_NSEG = 112
_N = 100000
_NW = 32
_C = 128
_NFULL = _N // _C
_TAIL = _N - _NFULL * _C
_TAIL_W = 31


def _sqrt16_batch(ss):
    ys = [plsc.bitcast(jnp.int32(0x5F3759DF)
                       - (plsc.bitcast(s, jnp.int32) >> 1), jnp.float32)
          for s in ss]
    hs = [0.5 * s for s in ss]
    for _ in range(2):
        ys = [y * (1.5 - h * y * y) for y, h in zip(ys, hs)]
    return [s * y for s, y in zip(ss, ys)]


def _colvec(xb, ob, off):
    for j0 in range(0, 64, 8):
        vs = [xb[j0 + j, pl.ds(off, 16)] for j in range(8)]
        for j in range(8):
            ob[j0 + j, pl.ds(off, 16)] = jnp.abs(vs[j])
    for k0 in range(0, 32, 8):
        gs = [[xb[64 + 3 * (k0 + k) + t, pl.ds(off, 16)] for t in range(3)]
              for k in range(8)]
        sq = [[g * g for g in row] for row in gs]
        ss = [r[0] + r[1] + r[2] for r in sq]
        rs = _sqrt16_batch(ss)
        for k in range(8):
            ob[64 + k0 + k, pl.ds(off, 16)] = rs[k]
    for k0 in range(0, 16, 8):
        gs = [[xb[160 + 5 * (k0 + k) + t, pl.ds(off, 16)] for t in range(5)]
              for k in range(8)]
        sq = [[g * g for g in row] for row in gs]
        ss = [((r[0] + r[1]) + (r[2] + r[3])) + r[4] for r in sq]
        rs = _sqrt16_batch(ss)
        for k in range(8):
            ob[96 + k0 + k, pl.ds(off, 16)] = rs[k]


def _body(x_hbm, xt_hbm, o_hbm, ot_hbm, xbuf, obuf, insem, outsem):
    wid = lax.axis_index("s") * 2 + lax.axis_index("c")
    nck = jnp.where(wid <= 12, 25, 24)

    def in_copy(k, slot):
        col0 = (wid + k * _NW) * _C
        return pltpu.make_async_copy(
            x_hbm.at[:, pl.ds(col0, _C)],
            xbuf.at[:, pl.ds(slot * _C, _C)],
            insem.at[slot])

    def out_copy(k, slot):
        col0 = (wid + k * _NW) * _C
        return pltpu.make_async_copy(
            obuf.at[:, pl.ds(slot * _C, _C)],
            o_hbm.at[:, pl.ds(col0, _C)],
            outsem.at[slot])

    in_copy(0, 0).start()

    @pl.loop(0, nck)
    def _chunk(k):
        slot = lax.rem(k, 2)

        @pl.when(k + 1 < nck)
        def _prefetch():
            in_copy(k + 1, 1 - slot).start()

        in_copy(k, slot).wait()

        @pl.when(k >= 2)
        def _drain():
            out_copy(k - 2, slot).wait()

        base = slot * _C

        @plsc.parallel_loop(0, _C // 16)
        def _cv(v):
            _colvec(xbuf, obuf, base + v * 16)

        out_copy(k, slot).start()

    out_copy(nck - 2, lax.rem(nck - 2, 2)).wait()
    out_copy(nck - 1, lax.rem(nck - 1, 2)).wait()

    @pl.when(wid == _TAIL_W)
    def _tail():
        pltpu.sync_copy(xt_hbm, xbuf.at[:, pl.ds(0, _C)])

        @plsc.parallel_loop(0, _TAIL // 16)
        def _cv(v):
            _colvec(xbuf, obuf, v * 16)

        pltpu.sync_copy(obuf.at[:, pl.ds(0, _C)], ot_hbm)


def kernel(features):
    size = features.shape[:-1]
    x = features.reshape(-1, _DIM)
    xt = x.T
    x_tail = jnp.pad(lax.slice(x, (_NFULL * _C, 0), (_N, _DIM)).T,
                     ((0, 0), (0, _C - _TAIL)))
    mesh = plsc.VectorSubcoreMesh(core_axis_name="c", subcore_axis_name="s",
                                  num_cores=2, num_subcores=16)
    out_t, out_tail = pl.kernel(
        _body,
        out_type=[
            jax.ShapeDtypeStruct((_NSEG, _N), jnp.float32),
            jax.ShapeDtypeStruct((_NSEG, _C), jnp.float32),
        ],
        mesh=mesh,
        scratch_types=[
            pltpu.VMEM((_DIM, 2 * _C), jnp.float32),
            pltpu.VMEM((_NSEG, 2 * _C), jnp.float32),
            pltpu.SemaphoreType.DMA((2,)),
            pltpu.SemaphoreType.DMA((2,)),
        ],
        compiler_params=pltpu.CompilerParams(needs_layout_passes=False),
    )(xt, x_tail)
    tail = lax.slice(out_tail, (0, 0), (_NSEG, _TAIL)).T
    out = lax.dynamic_update_slice(out_t.T, tail, (_NFULL * _C, 0))
    return out.reshape(size + (_NSEG,))
